# baseline (speedup 1.0000x reference)
import jax
import jax.numpy as jnp
from jax.experimental import pallas as pl
from jax.experimental.pallas import tpu as pltpu

_S_BLK = 512
_NBUF = 6


def _film_pipelined(ts_ref, x_hbm, tab_hbm, o_hbm, emb_buf, in_bufs, out_bufs,
                    emb_sem, in_sems, out_sems):
    B, S, D = x_hbm.shape
    nS = S // _S_BLK
    N = B * nS

    def x_view(i):
        return x_hbm.at[i // nS, pl.ds((i % nS) * _S_BLK, _S_BLK), :]

    def o_view(i):
        return o_hbm.at[i // nS, pl.ds((i % nS) * _S_BLK, _S_BLK), :]

    for b in range(B):
        pltpu.make_async_copy(tab_hbm.at[ts_ref[b]], emb_buf.at[b],
                              emb_sem).start()
    for k in range(_NBUF - 1):
        pltpu.make_async_copy(x_view(k), in_bufs.at[k], in_sems.at[k]).start()
    for b in range(B):
        pltpu.make_async_copy(tab_hbm.at[ts_ref[b]], emb_buf.at[b],
                              emb_sem).wait()

    for i in range(N):
        slot = i % _NBUF
        nxt = i + _NBUF - 1
        if nxt < N:
            pltpu.make_async_copy(x_view(nxt), in_bufs.at[nxt % _NBUF],
                                  in_sems.at[nxt % _NBUF]).start()
        pltpu.make_async_copy(x_view(i), in_bufs.at[slot], in_sems.at[slot]).wait()
        if i >= _NBUF:
            pltpu.make_async_copy(out_bufs.at[slot], o_view(i - _NBUF),
                                  out_sems.at[slot]).wait()
        b = i // nS
        shift = emb_buf[b, 0, :]
        scale = emb_buf[b, 1, :]
        out_bufs[slot] = jax.nn.gelu(in_bufs[slot] * scale + shift)
        pltpu.make_async_copy(out_bufs.at[slot], o_view(i), out_sems.at[slot]).start()

    for i in range(max(0, N - _NBUF), N):
        pltpu.make_async_copy(out_bufs.at[i % _NBUF], o_view(i),
                              out_sems.at[i % _NBUF]).wait()


def kernel(x, timestep, film_table):
    B, S, D = x.shape
    table3 = film_table.reshape(film_table.shape[0], 2, D)
    out = pl.pallas_call(
        _film_pipelined,
        in_specs=[
            pl.BlockSpec(memory_space=pltpu.MemorySpace.SMEM),
            pl.BlockSpec(memory_space=pl.MemorySpace.ANY),
            pl.BlockSpec(memory_space=pl.MemorySpace.ANY),
        ],
        out_specs=pl.BlockSpec(memory_space=pl.MemorySpace.ANY),
        out_shape=jax.ShapeDtypeStruct((B, S, D), x.dtype),
        scratch_shapes=[
            pltpu.VMEM((B, 2, D), jnp.float32),
            pltpu.VMEM((_NBUF, _S_BLK, D), jnp.float32),
            pltpu.VMEM((_NBUF, _S_BLK, D), jnp.float32),
            pltpu.SemaphoreType.DMA,
            pltpu.SemaphoreType.DMA((_NBUF,)),
            pltpu.SemaphoreType.DMA((_NBUF,)),
        ],
    )(timestep, x, table3)
    return out


# speedup vs baseline: 1.0143x; 1.0143x over previous
import jax
import jax.numpy as jnp
from jax.experimental import pallas as pl
from jax.experimental.pallas import tpu as pltpu

_S_BLK = 2048
_NBUF = 3


def _film_pipelined(ts_ref, x_hbm, tab_hbm, o_hbm, emb_buf, in_bufs, out_bufs,
                    emb_sem, in_sems, out_sems):
    B, S, D = x_hbm.shape
    nS = S // _S_BLK
    N = B * nS

    def x_view(i):
        return x_hbm.at[i // nS, pl.ds((i % nS) * _S_BLK, _S_BLK), :]

    def o_view(i):
        return o_hbm.at[i // nS, pl.ds((i % nS) * _S_BLK, _S_BLK), :]

    for b in range(B):
        pltpu.make_async_copy(tab_hbm.at[ts_ref[b]], emb_buf.at[b],
                              emb_sem).start()
    for k in range(_NBUF - 1):
        pltpu.make_async_copy(x_view(k), in_bufs.at[k], in_sems.at[k]).start()
    for b in range(B):
        pltpu.make_async_copy(tab_hbm.at[ts_ref[b]], emb_buf.at[b],
                              emb_sem).wait()

    for i in range(N):
        slot = i % _NBUF
        nxt = i + _NBUF - 1
        if nxt < N:
            pltpu.make_async_copy(x_view(nxt), in_bufs.at[nxt % _NBUF],
                                  in_sems.at[nxt % _NBUF]).start()
        pltpu.make_async_copy(x_view(i), in_bufs.at[slot], in_sems.at[slot]).wait()
        if i >= _NBUF:
            pltpu.make_async_copy(out_bufs.at[slot], o_view(i - _NBUF),
                                  out_sems.at[slot]).wait()
        b = i // nS
        shift = emb_buf[b, 0, :]
        scale = emb_buf[b, 1, :]
        out_bufs[slot] = jax.nn.gelu(in_bufs[slot] * scale + shift)
        pltpu.make_async_copy(out_bufs.at[slot], o_view(i), out_sems.at[slot]).start()

    for i in range(max(0, N - _NBUF), N):
        pltpu.make_async_copy(out_bufs.at[i % _NBUF], o_view(i),
                              out_sems.at[i % _NBUF]).wait()


def kernel(x, timestep, film_table):
    B, S, D = x.shape
    table3 = film_table.reshape(film_table.shape[0], 2, D)
    out = pl.pallas_call(
        _film_pipelined,
        in_specs=[
            pl.BlockSpec(memory_space=pltpu.MemorySpace.SMEM),
            pl.BlockSpec(memory_space=pl.MemorySpace.ANY),
            pl.BlockSpec(memory_space=pl.MemorySpace.ANY),
        ],
        out_specs=pl.BlockSpec(memory_space=pl.MemorySpace.ANY),
        out_shape=jax.ShapeDtypeStruct((B, S, D), x.dtype),
        scratch_shapes=[
            pltpu.VMEM((B, 2, D), jnp.float32),
            pltpu.VMEM((_NBUF, _S_BLK, D), jnp.float32),
            pltpu.VMEM((_NBUF, _S_BLK, D), jnp.float32),
            pltpu.SemaphoreType.DMA,
            pltpu.SemaphoreType.DMA((_NBUF,)),
            pltpu.SemaphoreType.DMA((_NBUF,)),
        ],
    )(timestep, x, table3)
    return out
